# baseline (speedup 1.0000x reference)
import functools

import jax
import jax.numpy as jnp
from jax import lax
from jax.experimental import pallas as pl
from jax.experimental.pallas import tpu as pltpu
from jax.experimental.pallas import tpu_sc as plsc

NUM_ACTIONS = 100
ACTION_DIM = 1000
PATCHDES_DIM = 256
EMB_DIM = 128
BATCH = 1024
NUM_PATCHES = 196
ACT_VOCAB = NUM_ACTIONS * ACTION_DIM
SEQ = NUM_PATCHES + NUM_ACTIONS

NTOT = BATCH * NUM_ACTIONS
NW = 32
PER_W = NTOT // NW
CHUNK = 128
NCHUNK = PER_W // CHUNK


def _sc_body(idx_hbm, dst_hbm, table_hbm, out_hbm, idx_v, dst_v, rows_v, sa, sb, sw):
    wid = lax.axis_index("s") * 2 + lax.axis_index("c")
    base = wid * PER_W
    pltpu.sync_copy(idx_hbm.at[pl.ds(base, PER_W)], idx_v)

    def gather(j, buf, sem):
        off = pl.multiple_of(j * CHUNK, CHUNK)
        return pltpu.async_copy(
            table_hbm.at[idx_v.at[pl.ds(off, CHUNK)]], rows_v.at[buf], sem
        )

    def gwait(j, buf, sem):
        off = pl.multiple_of(j * CHUNK, CHUNK)
        pltpu.make_async_copy(
            table_hbm.at[idx_v.at[pl.ds(off, CHUNK)]], rows_v.at[buf], sem
        ).wait()

    def scatter(j, buf):
        pltpu.async_copy(rows_v.at[buf], out_hbm.at[dst_v.at[j]], sw).wait()

    gather(0, 0, sa)
    pltpu.sync_copy(dst_hbm.at[wid], dst_v)

    def step(i, carry):
        gather(2 * i + 1, 1, sb)
        gwait(2 * i, 0, sa)
        scatter(2 * i, 0)

        gather(2 * i + 2, 0, sa)

        gwait(2 * i + 1, 1, sb)
        scatter(2 * i + 1, 1)
        return carry

    lax.fori_loop(0, (NCHUNK - 1) // 2, step, 0)
    gwait(NCHUNK - 1, 0, sa)
    scatter(NCHUNK - 1, 0)


@functools.lru_cache(maxsize=1)
def _sc_scatter():
    return pl.kernel(
        _sc_body,
        out_type=jax.ShapeDtypeStruct((BATCH * SEQ, EMB_DIM), jnp.float32),
        mesh=plsc.VectorSubcoreMesh(core_axis_name="c", subcore_axis_name="s"),
        scratch_types=[
            pltpu.VMEM((PER_W,), jnp.int32),
            pltpu.VMEM((NCHUNK, CHUNK), jnp.int32),
            pltpu.VMEM((2, CHUNK, EMB_DIM), jnp.float32),
            pltpu.SemaphoreType.DMA,
            pltpu.SemaphoreType.DMA,
            pltpu.SemaphoreType.DMA,
        ],
    )


TC_ROWS = 200


def _tc_body(a_ref, p_ref, w_ref, b_ref, ah_ref, o_ref):
    del a_ref
    x = p_ref[...].reshape(-1, PATCHDES_DIM)
    y = jnp.dot(x, w_ref[...], preferred_element_type=jnp.float32) + b_ref[...]
    o_ref[:, :NUM_PATCHES, :] = y.reshape(-1, NUM_PATCHES, EMB_DIM)
    o_ref[:, NUM_PATCHES:, :] = ah_ref[...]


def _tc_call(partial_out, patches, W_obs, b_obs, act_head, bsz=64):
    grid = BATCH // bsz
    return pl.pallas_call(
        _tc_body,
        grid=(grid,),
        in_specs=[
            pl.BlockSpec(memory_space=pltpu.MemorySpace.HBM),
            pl.BlockSpec((bsz, NUM_PATCHES, PATCHDES_DIM), lambda i: (i, 0, 0)),
            pl.BlockSpec((PATCHDES_DIM, EMB_DIM), lambda i: (0, 0)),
            pl.BlockSpec((1, EMB_DIM), lambda i: (0, 0)),
            pl.BlockSpec((bsz, TC_ROWS - NUM_PATCHES, EMB_DIM), lambda i: (i, 0, 0)),
        ],
        out_specs=pl.BlockSpec((bsz, TC_ROWS, EMB_DIM), lambda i: (i, 0, 0)),
        out_shape=jax.ShapeDtypeStruct((BATCH, SEQ, EMB_DIM), jnp.float32),
        input_output_aliases={0: 0},
        compiler_params=pltpu.CompilerParams(
            dimension_semantics=("arbitrary",),
        ),
    )(partial_out, patches, W_obs, b_obs, act_head)


def kernel(patches, action, W_obs, b_obs, emb_table):
    offsets = (jnp.arange(NUM_ACTIONS, dtype=action.dtype) * ACTION_DIM)[None, :]
    idx = (action + offsets).reshape(-1)
    p = jnp.arange(NTOT, dtype=jnp.int32)
    dst = (p // NUM_ACTIONS) * SEQ + NUM_PATCHES + (p % NUM_ACTIONS)
    dst3 = dst.reshape(NW, NCHUNK, CHUNK)
    partial_out = _sc_scatter()(idx, dst3, emb_table)
    partial_out = partial_out.reshape(BATCH, SEQ, EMB_DIM)
    act_head = lax.slice(
        partial_out, (0, NUM_PATCHES, 0), (BATCH, TC_ROWS, EMB_DIM)
    )
    return _tc_call(
        partial_out, patches, W_obs, b_obs.reshape(1, EMB_DIM), act_head
    )


# speedup vs baseline: 1.0285x; 1.0285x over previous
import functools

import jax
import jax.numpy as jnp
from jax import lax
from jax.experimental import pallas as pl
from jax.experimental.pallas import tpu as pltpu
from jax.experimental.pallas import tpu_sc as plsc

NUM_ACTIONS = 100
ACTION_DIM = 1000
PATCHDES_DIM = 256
EMB_DIM = 128
BATCH = 1024
NUM_PATCHES = 196
ACT_VOCAB = NUM_ACTIONS * ACTION_DIM
SEQ = NUM_PATCHES + NUM_ACTIONS

NTOT = BATCH * NUM_ACTIONS
NW = 32
PER_W = NTOT // NW
CHUNK = 128
NCHUNK = PER_W // CHUNK


def _sc_body(idx_hbm, dst_hbm, table_hbm, out_hbm, idx_v, dst_v, rows_v, sa, sb, sw):
    wid = lax.axis_index("s") * 2 + lax.axis_index("c")
    base = wid * PER_W
    pltpu.sync_copy(idx_hbm.at[pl.ds(base, PER_W)], idx_v)

    def gather(j, buf, sem):
        off = pl.multiple_of(j * CHUNK, CHUNK)
        return pltpu.async_copy(
            table_hbm.at[idx_v.at[pl.ds(off, CHUNK)]], rows_v.at[buf], sem
        )

    def gwait(j, buf, sem):
        off = pl.multiple_of(j * CHUNK, CHUNK)
        pltpu.make_async_copy(
            table_hbm.at[idx_v.at[pl.ds(off, CHUNK)]], rows_v.at[buf], sem
        ).wait()

    def scatter(j, buf):
        pltpu.async_copy(rows_v.at[buf], out_hbm.at[dst_v.at[j]], sw).wait()

    pltpu.sync_copy(dst_hbm.at[wid], dst_v)

    def step(j, carry):
        gather(j, 0, sa).wait()
        scatter(j, 0)
        return carry

    lax.fori_loop(0, NCHUNK, step, 0)


@functools.lru_cache(maxsize=1)
def _sc_scatter():
    return pl.kernel(
        _sc_body,
        out_type=jax.ShapeDtypeStruct((BATCH * SEQ, EMB_DIM), jnp.float32),
        mesh=plsc.VectorSubcoreMesh(core_axis_name="c", subcore_axis_name="s"),
        scratch_types=[
            pltpu.VMEM((PER_W,), jnp.int32),
            pltpu.VMEM((NCHUNK, CHUNK), jnp.int32),
            pltpu.VMEM((2, CHUNK, EMB_DIM), jnp.float32),
            pltpu.SemaphoreType.DMA,
            pltpu.SemaphoreType.DMA,
            pltpu.SemaphoreType.DMA,
        ],
    )


TC_ROWS = 200


def _tc_body(a_ref, p_ref, w_ref, b_ref, ah_ref, o_ref):
    del a_ref
    x = p_ref[...].reshape(-1, PATCHDES_DIM)
    y = jnp.dot(x, w_ref[...], preferred_element_type=jnp.float32) + b_ref[...]
    o_ref[:, :NUM_PATCHES, :] = y.reshape(-1, NUM_PATCHES, EMB_DIM)
    o_ref[:, NUM_PATCHES:, :] = ah_ref[...]


def _tc_call(partial_out, patches, W_obs, b_obs, act_head, bsz=64):
    grid = BATCH // bsz
    return pl.pallas_call(
        _tc_body,
        grid=(grid,),
        in_specs=[
            pl.BlockSpec(memory_space=pltpu.MemorySpace.HBM),
            pl.BlockSpec((bsz, NUM_PATCHES, PATCHDES_DIM), lambda i: (i, 0, 0)),
            pl.BlockSpec((PATCHDES_DIM, EMB_DIM), lambda i: (0, 0)),
            pl.BlockSpec((1, EMB_DIM), lambda i: (0, 0)),
            pl.BlockSpec((bsz, TC_ROWS - NUM_PATCHES, EMB_DIM), lambda i: (i, 0, 0)),
        ],
        out_specs=pl.BlockSpec((bsz, TC_ROWS, EMB_DIM), lambda i: (i, 0, 0)),
        out_shape=jax.ShapeDtypeStruct((BATCH, SEQ, EMB_DIM), jnp.float32),
        input_output_aliases={0: 0},
        compiler_params=pltpu.CompilerParams(
            dimension_semantics=("arbitrary",),
        ),
    )(partial_out, patches, W_obs, b_obs, act_head)


def kernel(patches, action, W_obs, b_obs, emb_table):
    offsets = (jnp.arange(NUM_ACTIONS, dtype=action.dtype) * ACTION_DIM)[None, :]
    idx = (action + offsets).reshape(-1)
    p = jnp.arange(NTOT, dtype=jnp.int32)
    dst = (p // NUM_ACTIONS) * SEQ + NUM_PATCHES + (p % NUM_ACTIONS)
    dst3 = dst.reshape(NW, NCHUNK, CHUNK)
    partial_out = _sc_scatter()(idx, dst3, emb_table)
    partial_out = partial_out.reshape(BATCH, SEQ, EMB_DIM)
    act_head = lax.slice(
        partial_out, (0, NUM_PATCHES, 0), (BATCH, TC_ROWS, EMB_DIM)
    )
    return _tc_call(
        partial_out, patches, W_obs, b_obs.reshape(1, EMB_DIM), act_head
    )
